# baseline (speedup 1.0000x reference)
import jax
import jax.numpy as jnp
from jax.experimental import pallas as pl
from jax.experimental.pallas import tpu as pltpu

B, D, N, H, RH = 8192, 16, 32, 256, 256
BB = 512


def _fused_kernel(x_ref, g_ref,
                  lw1_ref, lb1_ref, lw2_ref, lb2_ref, lw3_ref, lb3_ref,
                  rw1_ref, rb1_ref, rw2_ref, rb2_ref, rw3_ref, rb3_ref,
                  coeff_ref,
                  dxdt_ref, gates_ref):
    xb = x_ref[...]

    mlp_cols = []
    for n in range(N):
        h1 = jnp.maximum(
            jnp.dot(xb, lw1_ref[n], preferred_element_type=jnp.float32)
            + lb1_ref[n:n + 1, :], 0.0)
        h2 = jnp.maximum(
            jnp.dot(h1, lw2_ref[n], preferred_element_type=jnp.float32)
            + lb2_ref[n:n + 1, :], 0.0)
        o = jnp.sum(h2 * lw3_ref[n:n + 1, :], axis=1, keepdims=True) \
            + lb3_ref[n:n + 1, :]
        mlp_cols.append(o)
    mlp = jnp.concatenate(mlp_cols, axis=1)

    lane = jax.lax.broadcasted_iota(jnp.int32, (BB, N), 1)
    dx_cols = []
    for r in range(D):
        h1 = jnp.maximum(
            jnp.dot(xb, rw1_ref[r], preferred_element_type=jnp.float32)
            + rb1_ref[r:r + 1, :], 0.0)
        h2 = jnp.maximum(
            jnp.dot(h1, rw2_ref[r], preferred_element_type=jnp.float32)
            + rb2_ref[r:r + 1, :], 0.0)
        z = jnp.dot(h2, rw3_ref[r], preferred_element_type=jnp.float32) \
            + rb3_ref[r:r + 1, :] + g_ref[r]
        m = jnp.max(z, axis=1, keepdims=True)
        idx = jnp.min(jnp.where(z >= m, lane, N), axis=1, keepdims=True)
        onehot = (lane == idx).astype(jnp.float32)
        gates_ref[r] = onehot
        dx_cols.append(jnp.sum(onehot * coeff_ref[r:r + 1, :] * mlp,
                               axis=1, keepdims=True))
    dxdt_ref[...] = jnp.concatenate(dx_cols, axis=1)


def _gumbel_noise():
    u = jax.random.uniform(jax.random.key(1234), (D, B, N),
                           dtype=jnp.float32, minval=0.0, maxval=1.0)
    return -jnp.log(-jnp.log(jnp.clip(u, 1e-10, None)))


def kernel(X, lib_W1, lib_b1, lib_W2, lib_b2, lib_W3, lib_b3,
           r_W1, r_b1, r_W2, r_b2, r_W3, r_b3, coefficients):
    g = _gumbel_noise()
    lw3 = lib_W3[:, :, 0]

    def full(shape):
        return pl.BlockSpec(shape, lambda i: (0,) * len(shape))

    in_specs = [
        pl.BlockSpec((BB, D), lambda i: (i, 0)),
        pl.BlockSpec((D, BB, N), lambda i: (0, i, 0)),
        full((N, D, H)), full((N, H)), full((N, H, H)), full((N, H)),
        full((N, H)), full((N, 1)),
        full((D, D, RH)), full((D, RH)), full((D, RH, RH)), full((D, RH)),
        full((D, RH, N)), full((D, N)),
        full((D, N)),
    ]
    out_specs = [pl.BlockSpec((BB, D), lambda i: (i, 0)),
                 pl.BlockSpec((D, BB, N), lambda i: (0, i, 0))]
    out_shape = [jax.ShapeDtypeStruct((B, D), jnp.float32),
                 jax.ShapeDtypeStruct((D, B, N), jnp.float32)]
    dxdt, gates = pl.pallas_call(
        _fused_kernel,
        grid=(B // BB,),
        in_specs=in_specs,
        out_specs=out_specs,
        out_shape=out_shape,
    )(X, g, lib_W1, lib_b1, lib_W2, lib_b2, lw3, lib_b3,
      r_W1, r_b1, r_W2, r_b2, r_W3, r_b3, coefficients)
    return dxdt, gates


# speedup vs baseline: 1.2778x; 1.2778x over previous
import jax
import jax.numpy as jnp
from jax.experimental import pallas as pl
from jax.experimental.pallas import tpu as pltpu

B, D, N, H, RH = 8192, 16, 32, 256, 256
BB = 512


def _fused_kernel(x_ref, g_ref,
                  lw1_ref, lb1_ref, lw2_ref, lb2_ref, lw3_ref, lb3_ref,
                  rw1_ref, rb1_ref, rw2_ref, rb2_ref, rw3_ref, rb3_ref,
                  coeff_ref,
                  dxdt_ref, gates_ref):
    xb = x_ref[...]

    mlp_cols = []
    for n in range(N):
        h1 = jnp.maximum(
            jnp.dot(xb, lw1_ref[n], preferred_element_type=jnp.float32)
            + lb1_ref[n:n + 1, :], 0.0)
        h2 = jnp.maximum(
            jnp.dot(h1, lw2_ref[n], preferred_element_type=jnp.float32)
            + lb2_ref[n:n + 1, :], 0.0)
        o = jnp.dot(h2, lw3_ref[n], preferred_element_type=jnp.float32) \
            + lb3_ref[n:n + 1, :]
        mlp_cols.append(o)
    mlp = jnp.concatenate(mlp_cols, axis=1)

    lane = jax.lax.broadcasted_iota(jnp.int32, (BB, N), 1)
    dx_cols = []
    for r in range(D):
        h1 = jnp.maximum(
            jnp.dot(xb, rw1_ref[r], preferred_element_type=jnp.float32)
            + rb1_ref[r:r + 1, :], 0.0)
        h2 = jnp.maximum(
            jnp.dot(h1, rw2_ref[r], preferred_element_type=jnp.float32)
            + rb2_ref[r:r + 1, :], 0.0)
        z = jnp.dot(h2, rw3_ref[r], preferred_element_type=jnp.float32) \
            + rb3_ref[r:r + 1, :] + g_ref[r]
        m = jnp.max(z, axis=1, keepdims=True)
        idx = jnp.min(jnp.where(z >= m, lane, N), axis=1, keepdims=True)
        onehot = (lane == idx).astype(jnp.float32)
        gates_ref[r] = onehot
        dx_cols.append(jnp.dot(onehot * mlp, coeff_ref[:, r:r + 1],
                               preferred_element_type=jnp.float32))
    dxdt_ref[...] = jnp.concatenate(dx_cols, axis=1)


def _gumbel_noise():
    u = jax.random.uniform(jax.random.key(1234), (D, B, N),
                           dtype=jnp.float32, minval=0.0, maxval=1.0)
    return -jnp.log(-jnp.log(jnp.clip(u, 1e-10, None)))


def kernel(X, lib_W1, lib_b1, lib_W2, lib_b2, lib_W3, lib_b3,
           r_W1, r_b1, r_W2, r_b2, r_W3, r_b3, coefficients):
    g = _gumbel_noise()
    coeff_t = coefficients.T

    def full(shape):
        return pl.BlockSpec(shape, lambda i: (0,) * len(shape))

    in_specs = [
        pl.BlockSpec((BB, D), lambda i: (i, 0)),
        pl.BlockSpec((D, BB, N), lambda i: (0, i, 0)),
        full((N, D, H)), full((N, H)), full((N, H, H)), full((N, H)),
        full((N, H, 1)), full((N, 1)),
        full((D, D, RH)), full((D, RH)), full((D, RH, RH)), full((D, RH)),
        full((D, RH, N)), full((D, N)),
        full((N, D)),
    ]
    out_specs = [pl.BlockSpec((BB, D), lambda i: (i, 0)),
                 pl.BlockSpec((D, BB, N), lambda i: (0, i, 0))]
    out_shape = [jax.ShapeDtypeStruct((B, D), jnp.float32),
                 jax.ShapeDtypeStruct((D, B, N), jnp.float32)]
    dxdt, gates = pl.pallas_call(
        _fused_kernel,
        grid=(B // BB,),
        in_specs=in_specs,
        out_specs=out_specs,
        out_shape=out_shape,
    )(X, g, lib_W1, lib_b1, lib_W2, lib_b2, lib_W3, lib_b3,
      r_W1, r_b1, r_W2, r_b2, r_W3, r_b3, coeff_t)
    return dxdt, gates
